# baseline (speedup 1.0000x reference)
import functools

import jax
import jax.numpy as jnp
from jax import lax
from jax.experimental import pallas as pl
from jax.experimental.pallas import tpu as pltpu
from jax.experimental.pallas import tpu_sc as plsc

NC, NS = 2, 16
N = 10000
E = 320000
D = 128
DH = D // 2
DE = 16
K = 80
EPT = E // NS
NCHUNK = EPT // K
RPT = N // NS
ZROWS = 25


def _zero_fill(buf, rows, cols):
    zv = jnp.zeros((16,), jnp.float32)

    def row_body(i, _):
        def col_body(col, __):
            buf[i, pl.ds(col * 16, 16)] = zv
            return 0
        return lax.fori_loop(0, cols // 16, col_body, 0)

    lax.fori_loop(0, rows, row_body, 0)


NBUF = 5


def _seg_sum_body(with_ea, *refs):
    it = iter(refs)
    nxt = lambda: next(it)
    h0_hbm, h1_hbm, ei_hbm = nxt(), nxt(), nxt()
    ea_hbm = nxt() if with_ea else None
    out_s = nxt()
    out_a = nxt() if with_ea else None
    src_v, dst_v = nxt(), nxt()
    rows_v = tuple(nxt() for _ in range(NBUF))
    ea_v = nxt() if with_ea else None
    zbuf = nxt()
    zbuf_a = nxt() if with_ea else None
    acc_s = nxt()
    acc_a = nxt() if with_ea else None
    gsem = tuple(nxt() for _ in range(NBUF))
    ssem = tuple(nxt() for _ in range(NBUF))
    if with_ea:
        ldsem, easem = nxt(), nxt()

    c = lax.axis_index("c")
    s = lax.axis_index("s")

    _zero_fill(zbuf, ZROWS, DH)
    if with_ea:
        _zero_fill(zbuf_a, ZROWS, DE)
    for i in range(RPT // ZROWS):
        base = s * RPT + i * ZROWS
        pltpu.sync_copy(zbuf, acc_s.at[pl.ds(base, ZROWS)])
        if with_ea:
            pltpu.sync_copy(zbuf_a, acc_a.at[pl.ds(base, ZROWS)])
    plsc.subcore_barrier()

    e0 = s * EPT
    pltpu.sync_copy(ei_hbm.at[0, pl.ds(e0, EPT)], src_v)
    pltpu.sync_copy(ei_hbm.at[1, pl.ds(e0, EPT)], dst_v)

    NG = NCHUNK // NBUF

    def run_pass(h_hbm, do_ea, ea_lo=0, ea_hi=0):
        def idx(j):
            return pl.ds(j * K, K)

        def gather(j, b):
            pltpu.async_copy(h_hbm.at[src_v.at[idx(j)]], rows_v[b], gsem[b])

        def scatter(j, b):
            pltpu.async_copy(rows_v[b], acc_s.at[dst_v.at[idx(j)]], ssem[b],
                             add=True)

        def wait_g(b):
            pltpu.make_async_copy(h_hbm.at[src_v.at[idx(0)]], rows_v[b],
                                  gsem[b]).wait()

        def wait_s(b):
            pltpu.make_async_copy(rows_v[b], acc_s.at[dst_v.at[idx(0)]],
                                  ssem[b]).wait()

        def load_ea_slab(i):
            pltpu.async_copy(ea_hbm.at[pl.ds(e0 + i * NBUF * K, NBUF * K)],
                             ea_v, ldsem)

        def wait_ea_slab():
            pltpu.make_async_copy(ea_hbm.at[pl.ds(e0, NBUF * K)], ea_v,
                                  ldsem).wait()

        def scatter_ea(j, b):
            pltpu.async_copy(ea_v.at[pl.ds(b * K, K)],
                             acc_a.at[dst_v.at[idx(j)]], easem, add=True)

        def wait_ea_scatter():
            pltpu.make_async_copy(ea_v.at[pl.ds(0, K)],
                                  acc_a.at[dst_v.at[idx(0)]], easem).wait()

        for b in range(NBUF):
            gather(b, b)
        if do_ea:
            load_ea_slab(ea_lo)

        def group_body(i, _):
            jb = i * NBUF
            in_ea = (i >= ea_lo) & (i < ea_hi) if do_ea else False
            if do_ea:
                @pl.when(in_ea)
                def _():
                    wait_ea_slab()
            for b in range(NBUF):
                wait_g(b)
                if do_ea:
                    @pl.when(in_ea)
                    def _():
                        scatter_ea(jb + b, b)
                scatter(jb + b, b)
            for b in range(NBUF):
                wait_s(b)
                gather(jb + NBUF + b, b)
            if do_ea:
                @pl.when(in_ea & (i + 1 < ea_hi))
                def _():
                    for b in range(NBUF):
                        wait_ea_scatter()
                    load_ea_slab(i + 1)
            return 0

        lax.fori_loop(0, NG - 1, group_body, 0)

        jb = NCHUNK - NBUF
        last_ea = do_ea and ea_lo <= NG - 1 < ea_hi
        if last_ea:
            wait_ea_slab()
        for b in range(NBUF):
            wait_g(b)
            if last_ea:
                scatter_ea(jb + b, b)
            scatter(jb + b, b)
        for b in range(NBUF):
            wait_s(b)
        if last_ea:
            for b in range(NBUF):
                wait_ea_scatter()
        if do_ea and ea_hi < NG:
            for b in range(NBUF):
                wait_ea_scatter()

    @pl.when(c == 0)
    def _():
        run_pass(h0_hbm, with_ea, 0, NG // 2)

    @pl.when(c == 1)
    def _():
        run_pass(h1_hbm, with_ea, NG // 2, NG)

    plsc.subcore_barrier()

    pltpu.sync_copy(acc_s.at[pl.ds(s * RPT, RPT)], out_s.at[c, s])
    if with_ea:
        pltpu.sync_copy(acc_a.at[pl.ds(s * RPT, RPT)], out_a.at[c, s])


def _make_seg_sum(with_ea):
    mesh = plsc.VectorSubcoreMesh(
        core_axis_name="c", subcore_axis_name="s",
        num_cores=NC, num_subcores=NS)
    out_type = [jax.ShapeDtypeStruct((NC, NS, RPT, DH), jnp.float32)]
    scratch = [
        pltpu.VMEM((EPT,), jnp.int32),
        pltpu.VMEM((EPT,), jnp.int32),
    ]
    scratch += [pltpu.VMEM((K, DH), jnp.float32)] * NBUF
    if with_ea:
        out_type.append(jax.ShapeDtypeStruct((NC, NS, RPT, DE), jnp.float32))
        scratch.append(pltpu.VMEM((NBUF * K, DE), jnp.float32))
    scratch.append(pltpu.VMEM((ZROWS, DH), jnp.float32))
    if with_ea:
        scratch.append(pltpu.VMEM((ZROWS, DE), jnp.float32))
    scratch.append(pltpu.VMEM_SHARED((N, DH), jnp.float32))
    if with_ea:
        scratch.append(pltpu.VMEM_SHARED((N, DE), jnp.float32))
    scratch += [pltpu.SemaphoreType.DMA] * (2 * NBUF)
    if with_ea:
        scratch += [pltpu.SemaphoreType.DMA] * 2

    return pl.kernel(
        functools.partial(_seg_sum_body, with_ea),
        out_type=tuple(out_type),
        mesh=mesh,
        scratch_types=scratch,
        compiler_params=pltpu.CompilerParams(use_tc_tiling_on_sc=False),
        name=f"sc_seg_sum{'_ea' if with_ea else ''}",
    )


_make_seg_sum = functools.lru_cache(maxsize=None)(_make_seg_sum)


def _dense_body(last, h0_ref, h1_ref, s0_ref, s1_ref, a0_ref, a1_ref,
                wself_ref, wnbr_ref, wedge_ref, b_ref, g_ref, be_ref,
                *rest):
    if last:
        wh1_ref, bh1_ref, wh2_ref, bh2_ref, o_ref = rest
    else:
        o0_ref, o1_ref = rest
    h = jnp.concatenate([h0_ref[...], h1_ref[...]], axis=1)
    sacc = jnp.concatenate([s0_ref[...], s1_ref[...]], axis=1)
    aacc = a0_ref[...] + a1_ref[...]
    z = (jnp.dot(h, wself_ref[...], preferred_element_type=jnp.float32)
         + jnp.dot(sacc, wnbr_ref[...], preferred_element_type=jnp.float32)
         + jnp.dot(aacc, wedge_ref[...], preferred_element_type=jnp.float32)
         + b_ref[...])
    z = jnp.maximum(z, 0.0)
    mu = jnp.mean(z, axis=-1, keepdims=True)
    d = z - mu
    var = jnp.mean(d * d, axis=-1, keepdims=True)
    hn = d * lax.rsqrt(var + 1e-5) * g_ref[...] + be_ref[...]
    if last:
        t = jnp.maximum(
            jnp.dot(hn, wh1_ref[...], preferred_element_type=jnp.float32)
            + bh1_ref[...], 0.0)
        o_ref[...] = (jnp.dot(t, wh2_ref[...], preferred_element_type=jnp.float32)
                      + bh2_ref[...])
    else:
        o0_ref[...] = hn[:, :DH]
        o1_ref[...] = hn[:, DH:]


_ROWS_BLK = 2000


def _make_dense(last, interpret=False):
    grid = (N // _ROWS_BLK,)
    full = lambda r, c: pl.BlockSpec((r, c), lambda i: (0, 0))
    blk = lambda c: pl.BlockSpec((_ROWS_BLK, c), lambda i: (i, 0))
    in_specs = [
        blk(DH), blk(DH), blk(DH), blk(DH), blk(DE), blk(DE),
        full(D, D), full(D, D), full(DE, D), full(1, D), full(1, D), full(1, D),
    ]
    if last:
        in_specs += [full(D, D), full(1, D), full(D, 1), full(1, 1)]
        out_specs = pl.BlockSpec((_ROWS_BLK, 1), lambda i: (i, 0))
        out_shape = jax.ShapeDtypeStruct((N, 1), jnp.float32)
    else:
        out_specs = (blk(DH), blk(DH))
        out_shape = (jax.ShapeDtypeStruct((N, DH), jnp.float32),
                     jax.ShapeDtypeStruct((N, DH), jnp.float32))
    return pl.pallas_call(
        functools.partial(_dense_body, last),
        grid=grid,
        in_specs=in_specs,
        out_specs=out_specs,
        out_shape=out_shape,
        name=f"tc_dense{'_head' if last else ''}",
        interpret=interpret,
    )


def _prep_body(x_ref, o0_ref, o1_ref):
    v = x_ref[...]
    o0_ref[...] = v[:, :DH]
    o1_ref[...] = v[:, DH:]


_PBLK = 10

_prep = pl.pallas_call(
    _prep_body,
    grid=(_PBLK,),
    in_specs=[pl.BlockSpec((N // _PBLK, D), lambda i: (i, 0))],
    out_specs=(pl.BlockSpec((N // _PBLK, DH), lambda i: (i, 0)),
               pl.BlockSpec((N // _PBLK, DH), lambda i: (i, 0))),
    out_shape=(jax.ShapeDtypeStruct((N, DH), jnp.float32),
               jax.ShapeDtypeStruct((N, DH), jnp.float32)),
    name="tc_prep",
)


def kernel(x, edge_index, edge_attr, params):
    h0, h1 = _prep(x)
    ei4d = edge_index
    ea3d = edge_attr
    a0 = a1 = None
    out = None
    for i in range(3):
        if i == 0:
            s_parts, a_parts = _make_seg_sum(True)(h0, h1, ei4d, ea3d)
            a0 = a_parts[0].reshape(N, DE)
            a1 = a_parts[1].reshape(N, DE)
        else:
            (s_parts,) = _make_seg_sum(False)(h0, h1, ei4d)
        s0 = s_parts[0].reshape(N, DH)
        s1 = s_parts[1].reshape(N, DH)
        args = [
            h0, h1, s0, s1, a0, a1,
            params[f"W_self{i}"], params[f"W_nbr{i}"], params[f"W_edge{i}"],
            params[f"b{i}"].reshape(1, D),
            params[f"gamma{i}"].reshape(1, D),
            params[f"beta{i}"].reshape(1, D),
        ]
        if i == 2:
            args += [
                params["W_h1"], params["b_h1"].reshape(1, D),
                params["W_h2"], params["b_h2"].reshape(1, 1),
            ]
            out = _make_dense(True)(*args)
        else:
            h0, h1 = _make_dense(False)(*args)
    return out


# speedup vs baseline: 1.0126x; 1.0126x over previous
import functools

import jax
import jax.numpy as jnp
from jax import lax
from jax.experimental import pallas as pl
from jax.experimental.pallas import tpu as pltpu
from jax.experimental.pallas import tpu_sc as plsc

NC, NS = 2, 16
N = 10000
E = 320000
D = 128
DH = D // 2
DE = 16
K = 80
EPT = E // NS
NCHUNK = EPT // K
RPT = N // NS
ZROWS = 25


def _zero_fill(buf, rows, cols):
    zv = jnp.zeros((16,), jnp.float32)

    def row_body(i, _):
        def col_body(col, __):
            buf[i, pl.ds(col * 16, 16)] = zv
            return 0
        return lax.fori_loop(0, cols // 16, col_body, 0)

    lax.fori_loop(0, rows, row_body, 0)


NBUF = 5


def _seg_sum_body(with_ea, *refs):
    it = iter(refs)
    nxt = lambda: next(it)
    h0_hbm, h1_hbm, ei_hbm = nxt(), nxt(), nxt()
    ea_hbm = nxt() if with_ea else None
    out_s = nxt()
    out_a = nxt() if with_ea else None
    src_v, dst_v = nxt(), nxt()
    rows_v = tuple(nxt() for _ in range(NBUF))
    ea_v = nxt() if with_ea else None
    zbuf = nxt()
    zbuf_a = nxt() if with_ea else None
    acc_s = nxt()
    acc_a = nxt() if with_ea else None
    gsem = tuple(nxt() for _ in range(NBUF))
    ssem = tuple(nxt() for _ in range(NBUF))
    if with_ea:
        ldsem, easem = nxt(), nxt()

    c = lax.axis_index("c")
    s = lax.axis_index("s")

    _zero_fill(zbuf, ZROWS, DH)
    if with_ea:
        _zero_fill(zbuf_a, ZROWS, DE)
    for i in range(RPT // ZROWS):
        base = s * RPT + i * ZROWS
        pltpu.sync_copy(zbuf, acc_s.at[pl.ds(base, ZROWS)])
        if with_ea:
            pltpu.sync_copy(zbuf_a, acc_a.at[pl.ds(base, ZROWS)])
    plsc.subcore_barrier()

    e0 = s * EPT
    pltpu.sync_copy(ei_hbm.at[0, pl.ds(e0, EPT)], src_v)
    pltpu.sync_copy(ei_hbm.at[1, pl.ds(e0, EPT)], dst_v)

    def run_pass(h_hbm, do_ea):
        def idx(j):
            return pl.ds(j * K, K)

        def gather(j, b):
            pltpu.async_copy(h_hbm.at[src_v.at[idx(j)]], rows_v[b], gsem[b])

        def scatter(j, b):
            pltpu.async_copy(rows_v[b], acc_s.at[dst_v.at[idx(j)]], ssem[b],
                             add=True)

        def wait_g(b):
            pltpu.make_async_copy(h_hbm.at[src_v.at[idx(0)]], rows_v[b],
                                  gsem[b]).wait()

        def wait_s(b):
            pltpu.make_async_copy(rows_v[b], acc_s.at[dst_v.at[idx(0)]],
                                  ssem[b]).wait()

        def load_ea_slab(i):
            pltpu.async_copy(ea_hbm.at[pl.ds(e0 + i * NBUF * K, NBUF * K)],
                             ea_v, ldsem)

        def wait_ea_slab():
            pltpu.make_async_copy(ea_hbm.at[pl.ds(e0, NBUF * K)], ea_v,
                                  ldsem).wait()

        def scatter_ea(j, b):
            pltpu.async_copy(ea_v.at[pl.ds(b * K, K)],
                             acc_a.at[dst_v.at[idx(j)]], easem, add=True)

        def wait_ea_scatter():
            pltpu.make_async_copy(ea_v.at[pl.ds(0, K)],
                                  acc_a.at[dst_v.at[idx(0)]], easem).wait()

        for b in range(NBUF):
            gather(b, b)
        if do_ea:
            load_ea_slab(0)

        def group_body(i, _):
            jb = i * NBUF
            if do_ea:
                wait_ea_slab()
            for b in range(NBUF):
                wait_g(b)
                if do_ea:
                    scatter_ea(jb + b, b)
                scatter(jb + b, b)
            for b in range(NBUF):
                wait_s(b)
                gather(jb + NBUF + b, b)
            if do_ea:
                for b in range(NBUF):
                    wait_ea_scatter()
                load_ea_slab(i + 1)
            return 0

        lax.fori_loop(0, NCHUNK // NBUF - 1, group_body, 0)

        jb = NCHUNK - NBUF
        if do_ea:
            wait_ea_slab()
        for b in range(NBUF):
            wait_g(b)
            if do_ea:
                scatter_ea(jb + b, b)
            scatter(jb + b, b)
        for b in range(NBUF):
            wait_s(b)
        if do_ea:
            for b in range(NBUF):
                wait_ea_scatter()

    @pl.when(c == 0)
    def _():
        run_pass(h0_hbm, with_ea)

    @pl.when(c == 1)
    def _():
        run_pass(h1_hbm, False)

    plsc.subcore_barrier()

    pltpu.sync_copy(acc_s.at[pl.ds(s * RPT, RPT)], out_s.at[c, s])
    if with_ea:
        @pl.when(c == 0)
        def _():
            pltpu.sync_copy(acc_a.at[pl.ds(s * RPT, RPT)], out_a.at[s])


def _make_seg_sum(with_ea):
    mesh = plsc.VectorSubcoreMesh(
        core_axis_name="c", subcore_axis_name="s",
        num_cores=NC, num_subcores=NS)
    out_type = [jax.ShapeDtypeStruct((NC, NS, RPT, DH), jnp.float32)]
    scratch = [
        pltpu.VMEM((EPT,), jnp.int32),
        pltpu.VMEM((EPT,), jnp.int32),
    ]
    scratch += [pltpu.VMEM((K, DH), jnp.float32)] * NBUF
    if with_ea:
        out_type.append(jax.ShapeDtypeStruct((NS, RPT, DE), jnp.float32))
        scratch.append(pltpu.VMEM((NBUF * K, DE), jnp.float32))
    scratch.append(pltpu.VMEM((ZROWS, DH), jnp.float32))
    if with_ea:
        scratch.append(pltpu.VMEM((ZROWS, DE), jnp.float32))
    scratch.append(pltpu.VMEM_SHARED((N, DH), jnp.float32))
    if with_ea:
        scratch.append(pltpu.VMEM_SHARED((N, DE), jnp.float32))
    scratch += [pltpu.SemaphoreType.DMA] * (2 * NBUF)
    if with_ea:
        scratch += [pltpu.SemaphoreType.DMA] * 2

    return pl.kernel(
        functools.partial(_seg_sum_body, with_ea),
        out_type=tuple(out_type),
        mesh=mesh,
        scratch_types=scratch,
        compiler_params=pltpu.CompilerParams(use_tc_tiling_on_sc=False),
        name=f"sc_seg_sum{'_ea' if with_ea else ''}",
    )


_make_seg_sum = functools.lru_cache(maxsize=None)(_make_seg_sum)


def _dense_body(last, h0_ref, h1_ref, s0_ref, s1_ref, a_ref,
                wself_ref, wnbr_ref, wedge_ref, b_ref, g_ref, be_ref,
                *rest):
    if last:
        wh1_ref, bh1_ref, wh2_ref, bh2_ref, o_ref = rest
    else:
        o0_ref, o1_ref = rest
    h = jnp.concatenate([h0_ref[...], h1_ref[...]], axis=1)
    sacc = jnp.concatenate([s0_ref[...], s1_ref[...]], axis=1)
    z = (jnp.dot(h, wself_ref[...], preferred_element_type=jnp.float32)
         + jnp.dot(sacc, wnbr_ref[...], preferred_element_type=jnp.float32)
         + jnp.dot(a_ref[...], wedge_ref[...], preferred_element_type=jnp.float32)
         + b_ref[...])
    z = jnp.maximum(z, 0.0)
    mu = jnp.mean(z, axis=-1, keepdims=True)
    d = z - mu
    var = jnp.mean(d * d, axis=-1, keepdims=True)
    hn = d * lax.rsqrt(var + 1e-5) * g_ref[...] + be_ref[...]
    if last:
        t = jnp.maximum(
            jnp.dot(hn, wh1_ref[...], preferred_element_type=jnp.float32)
            + bh1_ref[...], 0.0)
        o_ref[...] = (jnp.dot(t, wh2_ref[...], preferred_element_type=jnp.float32)
                      + bh2_ref[...])
    else:
        o0_ref[...] = hn[:, :DH]
        o1_ref[...] = hn[:, DH:]


_ROWS_BLK = 2000


def _make_dense(last, interpret=False):
    grid = (N // _ROWS_BLK,)
    full = lambda r, c: pl.BlockSpec((r, c), lambda i: (0, 0))
    blk = lambda c: pl.BlockSpec((_ROWS_BLK, c), lambda i: (i, 0))
    in_specs = [
        blk(DH), blk(DH), blk(DH), blk(DH), blk(DE),
        full(D, D), full(D, D), full(DE, D), full(1, D), full(1, D), full(1, D),
    ]
    if last:
        in_specs += [full(D, D), full(1, D), full(D, 1), full(1, 1)]
        out_specs = pl.BlockSpec((_ROWS_BLK, 1), lambda i: (i, 0))
        out_shape = jax.ShapeDtypeStruct((N, 1), jnp.float32)
    else:
        out_specs = (blk(DH), blk(DH))
        out_shape = (jax.ShapeDtypeStruct((N, DH), jnp.float32),
                     jax.ShapeDtypeStruct((N, DH), jnp.float32))
    return pl.pallas_call(
        functools.partial(_dense_body, last),
        grid=grid,
        in_specs=in_specs,
        out_specs=out_specs,
        out_shape=out_shape,
        name=f"tc_dense{'_head' if last else ''}",
        interpret=interpret,
    )


def _prep_body(x_ref, o0_ref, o1_ref):
    v = x_ref[...]
    o0_ref[...] = v[:, :DH]
    o1_ref[...] = v[:, DH:]


_PBLK = 10

_prep = pl.pallas_call(
    _prep_body,
    grid=(_PBLK,),
    in_specs=[pl.BlockSpec((N // _PBLK, D), lambda i: (i, 0))],
    out_specs=(pl.BlockSpec((N // _PBLK, DH), lambda i: (i, 0)),
               pl.BlockSpec((N // _PBLK, DH), lambda i: (i, 0))),
    out_shape=(jax.ShapeDtypeStruct((N, DH), jnp.float32),
               jax.ShapeDtypeStruct((N, DH), jnp.float32)),
    name="tc_prep",
)


def kernel(x, edge_index, edge_attr, params):
    h0, h1 = _prep(x)
    ei4d = edge_index
    ea3d = edge_attr
    a = None
    out = None
    for i in range(3):
        if i == 0:
            s_parts, a_parts = _make_seg_sum(True)(h0, h1, ei4d, ea3d)
            a = a_parts.reshape(N, DE)
        else:
            (s_parts,) = _make_seg_sum(False)(h0, h1, ei4d)
        s0 = s_parts[0].reshape(N, DH)
        s1 = s_parts[1].reshape(N, DH)
        args = [
            h0, h1, s0, s1, a,
            params[f"W_self{i}"], params[f"W_nbr{i}"], params[f"W_edge{i}"],
            params[f"b{i}"].reshape(1, D),
            params[f"gamma{i}"].reshape(1, D),
            params[f"beta{i}"].reshape(1, D),
        ]
        if i == 2:
            args += [
                params["W_h1"], params["b_h1"].reshape(1, D),
                params["W_h2"], params["b_h2"].reshape(1, 1),
            ]
            out = _make_dense(True)(*args)
        else:
            h0, h1 = _make_dense(False)(*args)
    return out
